# baseline (speedup 1.0000x reference)
import functools

import jax
import jax.numpy as jnp
from jax import lax
from jax.experimental import pallas as pl
from jax.experimental.pallas import tpu as pltpu
from jax.experimental.pallas import tpu_sc as plsc

_NUM_CORES = 2
_NUM_SUBCORES = 16
_NUM_WORKERS = _NUM_CORES * _NUM_SUBCORES
_CHUNK = 64


def kernel(inputs, input_table, position_table):
    batch, seqlen = inputs.shape
    vocab, embed = input_table.shape
    n = batch * seqlen
    rpw = n // _NUM_WORKERS
    n_chunks = rpw // _CHUNK
    wpb = seqlen // rpw

    mesh = plsc.VectorSubcoreMesh(
        core_axis_name="c", subcore_axis_name="s",
        num_cores=_NUM_CORES, num_subcores=_NUM_SUBCORES)

    @functools.partial(
        pl.kernel,
        out_type=jax.ShapeDtypeStruct((batch, seqlen, embed), jnp.float32),
        mesh=mesh,
        scratch_types=[
            pltpu.VMEM((rpw,), jnp.int32),
            pltpu.VMEM((2, rpw // 2), jnp.int32),
            pltpu.VMEM((rpw, embed), jnp.float32),
            pltpu.VMEM_SHARED((_NUM_SUBCORES * rpw, embed), jnp.float32),
            pltpu.SemaphoreType.DMA,
            pltpu.SemaphoreType.DMA,
            pltpu.SemaphoreType.DMA,
        ],
    )
    def emb_kernel(idx_hbm, tab_hbm, pos_hbm, out_hbm,
                   idx_v, scat_v, rows_v, shared, sem, sem2, sem3):
        c = lax.axis_index("c")
        s = lax.axis_index("s")
        wid = s * _NUM_CORES + c
        b = wid // wpb
        col = (wid % wpb) * rpw
        base = s * rpw
        half = rpw // 2
        cp_idx = pltpu.async_copy(idx_hbm.at[b, pl.ds(col, rpw)], idx_v, sem)
        cp_pos = [
            pltpu.async_copy(
                pos_hbm.at[pl.ds(col + h * half, half)],
                shared.at[pl.ds(base + h * half, half)], [sem2, sem3][h])
            for h in range(2)
        ]
        lanes = lax.iota(jnp.int32, 16)
        for h in range(2):
            for k in range(half // 16):
                scat_v[h, pl.ds(k * 16, 16)] = lanes + (
                    base + h * half + k * 16)
        cp_idx.wait()
        gathers = [
            pltpu.async_copy(
                tab_hbm.at[idx_v.at[pl.ds(j * _CHUNK, _CHUNK)]],
                rows_v.at[pl.ds(j * _CHUNK, _CHUNK)], sem)
            for j in range(n_chunks)
        ]
        outs = []
        for h in range(2):
            for j in range(2 * h, 2 * h + 2):
                gathers[j].wait()
            cp_pos[h].wait()
            pltpu.sync_copy(
                rows_v.at[pl.ds(h * half, half)],
                shared.at[scat_v.at[h]], add=True)
            outs.append(pltpu.async_copy(
                shared.at[pl.ds(base + h * half, half)],
                out_hbm.at[b, pl.ds(col + h * half, half)], sem))
        for o in outs:
            o.wait()

    return emb_kernel(inputs, input_table, position_table)


# speedup vs baseline: 1.0171x; 1.0171x over previous
import functools

import jax
import jax.numpy as jnp
from jax import lax
from jax.experimental import pallas as pl
from jax.experimental.pallas import tpu as pltpu
from jax.experimental.pallas import tpu_sc as plsc

_NUM_CORES = 2
_NUM_SUBCORES = 16
_NUM_WORKERS = _NUM_CORES * _NUM_SUBCORES
_CHUNK = 64


def kernel(inputs, input_table, position_table):
    batch, seqlen = inputs.shape
    vocab, embed = input_table.shape
    n = batch * seqlen
    rpw = n // _NUM_WORKERS
    n_chunks = rpw // _CHUNK
    wpb = seqlen // rpw

    mesh = plsc.VectorSubcoreMesh(
        core_axis_name="c", subcore_axis_name="s",
        num_cores=_NUM_CORES, num_subcores=_NUM_SUBCORES)

    @functools.partial(
        pl.kernel,
        out_type=jax.ShapeDtypeStruct((batch, seqlen, embed), jnp.float32),
        mesh=mesh,
        scratch_types=[
            pltpu.VMEM((rpw,), jnp.int32),
            pltpu.VMEM((n_chunks, _CHUNK), jnp.int32),
            pltpu.VMEM((rpw, embed), jnp.float32),
            pltpu.VMEM_SHARED((_NUM_SUBCORES * rpw, embed), jnp.float32),
            pltpu.SemaphoreType.DMA,
            pltpu.SemaphoreType.DMA,
            pltpu.SemaphoreType.DMA,
        ],
    )
    def emb_kernel(idx_hbm, tab_hbm, pos_hbm, out_hbm,
                   idx_v, scat_v, rows_v, shared, sem, sem2, sem3):
        c = lax.axis_index("c")
        s = lax.axis_index("s")
        wid = s * _NUM_CORES + c
        b = wid // wpb
        col = (wid % wpb) * rpw
        base = s * rpw
        cp_pos = pltpu.async_copy(
            pos_hbm.at[pl.ds(col, rpw)], shared.at[pl.ds(base, rpw)], sem2)
        cp_idx = pltpu.async_copy(idx_hbm.at[b, pl.ds(col, rpw)], idx_v, sem)
        lanes = lax.iota(jnp.int32, 16)
        for j in range(n_chunks):
            for k in range(_CHUNK // 16):
                scat_v[j, pl.ds(k * 16, 16)] = lanes + (
                    base + j * _CHUNK + k * 16)
        cp_idx.wait()
        gathers = [
            pltpu.async_copy(
                tab_hbm.at[idx_v.at[pl.ds(j * _CHUNK, _CHUNK)]],
                rows_v.at[pl.ds(j * _CHUNK, _CHUNK)], sem)
            for j in range(n_chunks)
        ]
        cp_pos.wait()
        outs = []
        for j in range(n_chunks):
            gathers[j].wait()
            pltpu.sync_copy(
                rows_v.at[pl.ds(j * _CHUNK, _CHUNK)],
                shared.at[scat_v.at[j]], add=True)
            outs.append(pltpu.async_copy(
                shared.at[pl.ds(base + j * _CHUNK, _CHUNK)],
                out_hbm.at[b, pl.ds(col + j * _CHUNK, _CHUNK)], sem3))
        for o in outs:
            o.wait()

    return emb_kernel(inputs, input_table, position_table)
